# Initial kernel scaffold; baseline (speedup 1.0000x reference)
import functools

import jax
import jax.numpy as jnp
from jax import lax
from jax.experimental import pallas as pl
from jax.experimental.pallas import tpu as pltpu
from jax.experimental.pallas import tpu_sc as plsc

N = 10000
E = 320000
D = 128
DE = 16
U = 128

NC = 2
NS = 16
NW = NC * NS
CH = 128
NCHUNK = -(-E // (NW * CH))
EPAD = NW * CH * NCHUNK
PERW = CH * NCHUNK
RPS = N // NS

EBLK = 512
NBLK = 1000


def _sc_mesh():
    return plsc.VectorSubcoreMesh(core_axis_name="c", subcore_axis_name="s")


def _sc_gather(qtab, ktab, dst, src):
    ot = (jax.ShapeDtypeStruct((EPAD, U), jnp.float32),
          jax.ShapeDtypeStruct((EPAD, U), jnp.float32))

    @functools.partial(
        pl.kernel, mesh=_sc_mesh(), out_type=ot,
        scratch_types=[pltpu.VMEM((CH,), jnp.int32),
                       pltpu.VMEM((CH,), jnp.int32),
                       pltpu.VMEM((CH, U), jnp.float32),
                       pltpu.VMEM((CH, U), jnp.float32),
                       pltpu.SemaphoreType.DMA,
                       pltpu.SemaphoreType.DMA])
    def body(qtab_h, ktab_h, dst_h, src_h, qd_h, xs_h,
             idxq, idxk, bufq, bufk, sq, sk):
        wid = lax.axis_index("s") * NC + lax.axis_index("c")
        base = wid * PERW

        @pl.loop(0, NCHUNK)
        def _(j):
            off = base + j * CH
            pltpu.sync_copy(dst_h.at[pl.ds(off, CH)], idxq)
            pltpu.sync_copy(src_h.at[pl.ds(off, CH)], idxk)
            cq = pltpu.async_copy(qtab_h.at[idxq], bufq, sq)
            ck = pltpu.async_copy(ktab_h.at[idxk], bufk, sk)
            cq.wait()
            ck.wait()
            pltpu.sync_copy(bufq, qd_h.at[pl.ds(off, CH)])
            pltpu.sync_copy(bufk, xs_h.at[pl.ds(off, CH)])

    return body(qtab, ktab, dst, src)


def _sc_scatter(wv, ep, dst, z128, z16):
    ot = (jax.ShapeDtypeStruct((NC, N, U), jnp.float32),
          jax.ShapeDtypeStruct((NC, N, 16), jnp.float32))

    @functools.partial(
        pl.kernel, mesh=_sc_mesh(), out_type=ot,
        scratch_types=[pltpu.VMEM((CH,), jnp.int32),
                       pltpu.VMEM((CH, U), jnp.float32),
                       pltpu.VMEM((CH, 16), jnp.float32),
                       pltpu.VMEM_SHARED((N, U), jnp.float32),
                       pltpu.VMEM_SHARED((N, 16), jnp.float32)])
    def body(wv_h, ep_h, dst_h, z128_h, z16_h, acc_h, ssum_h,
             idx, bufw, bufe, accsh, ssumsh):
        cid = lax.axis_index("c")
        sid = lax.axis_index("s")
        wid = sid * NC + cid
        rbase = sid * RPS
        pltpu.sync_copy(z128_h.at[pl.ds(rbase, RPS)],
                        accsh.at[pl.ds(rbase, RPS)])
        pltpu.sync_copy(z16_h.at[pl.ds(rbase, RPS)],
                        ssumsh.at[pl.ds(rbase, RPS)])
        plsc.subcore_barrier()
        base = wid * PERW

        @pl.loop(0, NCHUNK)
        def _(j):
            off = base + j * CH
            pltpu.sync_copy(dst_h.at[pl.ds(off, CH)], idx)
            pltpu.sync_copy(wv_h.at[pl.ds(off, CH)], bufw)
            pltpu.sync_copy(ep_h.at[pl.ds(off, CH)], bufe)
            pltpu.sync_copy(bufw, accsh.at[idx], add=True)
            pltpu.sync_copy(bufe, ssumsh.at[idx], add=True)

        plsc.subcore_barrier()
        pltpu.sync_copy(accsh.at[pl.ds(rbase, RPS)],
                        acc_h.at[cid, pl.ds(rbase, RPS)])
        pltpu.sync_copy(ssumsh.at[pl.ds(rbase, RPS)],
                        ssum_h.at[cid, pl.ds(rbase, RPS)])

    return body(wv, ep, dst, z128, z16)


def _tc_node_transform(xin, Wq, Wkn):
    def body(x_ref, wq_ref, wk_ref, q_ref, k_ref):
        xb = x_ref[...]
        q_ref[...] = jnp.dot(xb, wq_ref[...],
                             preferred_element_type=jnp.float32)
        k_ref[...] = jnp.dot(xb, wk_ref[...],
                             preferred_element_type=jnp.float32)

    return pl.pallas_call(
        body,
        grid=(N // NBLK,),
        in_specs=[pl.BlockSpec((NBLK, D), lambda i: (i, 0)),
                  pl.BlockSpec((D, U), lambda i: (0, 0)),
                  pl.BlockSpec((D, U), lambda i: (0, 0))],
        out_specs=[pl.BlockSpec((NBLK, U), lambda i: (i, 0)),
                   pl.BlockSpec((NBLK, U), lambda i: (i, 0))],
        out_shape=[jax.ShapeDtypeStruct((N, U), jnp.float32),
                   jax.ShapeDtypeStruct((N, U), jnp.float32)],
    )(xin, Wq, Wkn)


def _tc_edge_math(qd, xs, eap, Wke, aflat, Ssel, Bsel, hmask):
    def body(qd_ref, xs_ref, ea_ref, wke_ref, a_ref, s_ref, b_ref, hm_ref,
             wv_ref, ep_ref):
        i = pl.program_id(0)
        k = xs_ref[...] + jnp.dot(ea_ref[...], wke_ref[...],
                                  preferred_element_type=jnp.float32)
        z = qd_ref[...] + k
        z = jnp.where(z >= 0.0, z, 0.2 * z)
        zw = z * a_ref[...]
        logits = jnp.dot(zw, s_ref[...], preferred_element_type=jnp.float32)
        e = jnp.exp(jnp.minimum(logits, 50.0)) * hm_ref[...]
        rowid = i * EBLK + lax.broadcasted_iota(jnp.int32, (EBLK, 1), 0)
        e = jnp.where(rowid < E, e, 0.0)
        ep_ref[...] = e
        wv_ref[...] = k * jnp.dot(e, b_ref[...],
                                  preferred_element_type=jnp.float32)

    return pl.pallas_call(
        body,
        grid=(EPAD // EBLK,),
        in_specs=[pl.BlockSpec((EBLK, U), lambda i: (i, 0)),
                  pl.BlockSpec((EBLK, U), lambda i: (i, 0)),
                  pl.BlockSpec((EBLK, DE), lambda i: (i, 0)),
                  pl.BlockSpec((DE, U), lambda i: (0, 0)),
                  pl.BlockSpec((1, U), lambda i: (0, 0)),
                  pl.BlockSpec((U, 16), lambda i: (0, 0)),
                  pl.BlockSpec((16, U), lambda i: (0, 0)),
                  pl.BlockSpec((1, 16), lambda i: (0, 0))],
        out_specs=[pl.BlockSpec((EBLK, U), lambda i: (i, 0)),
                   pl.BlockSpec((EBLK, 16), lambda i: (i, 0))],
        out_shape=[jax.ShapeDtypeStruct((EPAD, U), jnp.float32),
                   jax.ShapeDtypeStruct((EPAD, 16), jnp.float32)],
    )(qd, xs, eap, Wke, aflat, Ssel, Bsel, hmask)


def _tc_merge_next(acc, ssum, xin, Wna, Wnb, bn, Bsel):
    def body(a0_ref, a1_ref, s0_ref, s1_ref, x_ref, wa_ref, wb_ref, b_ref,
             bs_ref, h_ref):
        accw = a0_ref[0] + a1_ref[0]
        ssumw = s0_ref[0] + s1_ref[0]
        denom = jnp.dot(ssumw, bs_ref[...],
                        preferred_element_type=jnp.float32) + 1e-9
        pooled = accw / denom
        h = (jnp.dot(x_ref[...], wa_ref[...],
                     preferred_element_type=jnp.float32)
             + jnp.dot(pooled, wb_ref[...],
                       preferred_element_type=jnp.float32)
             + b_ref[...])
        h_ref[...] = jnp.maximum(h, 0.0)

    return pl.pallas_call(
        body,
        grid=(N // NBLK,),
        in_specs=[pl.BlockSpec((1, NBLK, U), lambda i: (0, i, 0)),
                  pl.BlockSpec((1, NBLK, U), lambda i: (1, i, 0)),
                  pl.BlockSpec((1, NBLK, 16), lambda i: (0, i, 0)),
                  pl.BlockSpec((1, NBLK, 16), lambda i: (1, i, 0)),
                  pl.BlockSpec((NBLK, U), lambda i: (i, 0)),
                  pl.BlockSpec((U, U), lambda i: (0, 0)),
                  pl.BlockSpec((U, U), lambda i: (0, 0)),
                  pl.BlockSpec((1, U), lambda i: (0, 0)),
                  pl.BlockSpec((16, U), lambda i: (0, 0))],
        out_specs=pl.BlockSpec((NBLK, U), lambda i: (i, 0)),
        out_shape=jax.ShapeDtypeStruct((N, U), jnp.float32),
    )(acc, acc, ssum, ssum, xin, Wna, Wnb, bn, Bsel)


def _tc_readout(h2, Wout, bout):
    def body(h_ref, w_ref, b_ref, o_ref):
        o_ref[...] = (jnp.dot(h_ref[...], w_ref[...],
                              preferred_element_type=jnp.float32)
                      + b_ref[...])

    return pl.pallas_call(
        body,
        grid=(N // NBLK,),
        in_specs=[pl.BlockSpec((NBLK, U), lambda i: (i, 0)),
                  pl.BlockSpec((U, 1), lambda i: (0, 0)),
                  pl.BlockSpec((1, 1), lambda i: (0, 0))],
        out_specs=pl.BlockSpec((NBLK, 1), lambda i: (i, 0)),
        out_shape=jax.ShapeDtypeStruct((N, 1), jnp.float32),
    )(h2, Wout, bout)


def _gat_layer(xin, Wq, Wk, aflat, Ssel, Bsel, hmask, Wn, bn,
               dstp, srcp, eap, z128, z16, din):
    Wkn = Wk[:din]
    Wke = Wk[din:]
    qtab, ktab = _tc_node_transform(xin, Wq, Wkn)
    qd, xs = _sc_gather(qtab, ktab, dstp, srcp)
    wv, ep = _tc_edge_math(qd, xs, eap, Wke, aflat, Ssel, Bsel, hmask)
    acc, ssum = _sc_scatter(wv, ep, dstp, z128, z16)
    Wna = Wn[:din]
    Wnb = Wn[din:]
    return _tc_merge_next(acc, ssum, xin, Wna, Wnb, bn.reshape(1, U), Bsel)


def kernel(x, edge_index, edge_attr, Wq1, Wk1, a1, Wn1, bn1,
           Wq2, Wk2, a2, Wn2, bn2, Wout, bout):
    src = edge_index[0]
    dst = edge_index[1]
    pad = EPAD - E
    dstp = jnp.concatenate([dst, jnp.zeros((pad,), dst.dtype)])
    srcp = jnp.concatenate([src, jnp.zeros((pad,), src.dtype)])
    eap = jnp.concatenate(
        [edge_attr, jnp.zeros((pad, DE), edge_attr.dtype)], axis=0)
    z128 = jnp.zeros((N, U), jnp.float32)
    z16 = jnp.zeros((N, 16), jnp.float32)

    lanes = jnp.arange(16)
    head1 = jnp.arange(U) // 32
    S1 = (head1[:, None] == lanes[None, :]).astype(jnp.float32)
    B1 = S1.T
    hm1 = (lanes < 4).astype(jnp.float32).reshape(1, 16)
    S2 = (jnp.zeros((U, 1), jnp.int32) == lanes[None, :]).astype(jnp.float32)
    B2 = S2.T
    hm2 = (lanes < 1).astype(jnp.float32).reshape(1, 16)

    a1flat = a1.reshape(1, U)
    a2flat = a2.reshape(1, U)

    h1 = _gat_layer(x, Wq1, Wk1, a1flat, S1, B1, hm1, Wn1, bn1,
                    dstp, srcp, eap, z128, z16, D)
    h2 = _gat_layer(h1, Wq2, Wk2, a2flat, S2, B2, hm2, Wn2, bn2,
                    dstp, srcp, eap, z128, z16, U)
    return _tc_readout(h2, Wout, bout.reshape(1, 1))


# speedup vs baseline: 5.7729x; 5.7729x over previous
import dataclasses
import functools

import jax
import jax.numpy as jnp
from jax import lax
from jax.experimental import pallas as pl
from jax.experimental.pallas import tpu as pltpu
from jax.experimental.pallas import tpu_sc as plsc

N = 10000
E = 320000
D = 128
DE = 16
U = 128

NC = 2
NS = 16
NW = NC * NS
CH = 128
NCHUNK = -(-E // (NW * CH))
EPAD = NW * CH * NCHUNK
PERW = CH * NCHUNK
NPAD = 10240
RPS = NPAD // NS

EBLK = 512
NBLKM = 1280
NBLK = 1000


def _sc_mesh():
    return plsc.VectorSubcoreMesh(core_axis_name="c", subcore_axis_name="s")


def _sc_gather(qtab, ktab, dst, src):
    ot = (jax.ShapeDtypeStruct((EPAD, U), jnp.float32),
          jax.ShapeDtypeStruct((EPAD, U), jnp.float32))

    @functools.partial(
        pl.kernel, mesh=_sc_mesh(), out_type=ot,
        scratch_types=[pltpu.VMEM((CH,), jnp.int32),
                       pltpu.VMEM((CH,), jnp.int32),
                       pltpu.VMEM((CH, U), jnp.float32),
                       pltpu.VMEM((CH, U), jnp.float32),
                       pltpu.SemaphoreType.DMA,
                       pltpu.SemaphoreType.DMA])
    def body(qtab_h, ktab_h, dst_h, src_h, qd_h, xs_h,
             idxq, idxk, bufq, bufk, sq, sk):
        wid = lax.axis_index("s") * NC + lax.axis_index("c")
        base = wid * PERW

        @pl.loop(0, NCHUNK)
        def _(j):
            off = base + j * CH
            pltpu.sync_copy(dst_h.at[pl.ds(off, CH)], idxq)
            pltpu.sync_copy(src_h.at[pl.ds(off, CH)], idxk)
            cq = pltpu.async_copy(qtab_h.at[idxq], bufq, sq)
            ck = pltpu.async_copy(ktab_h.at[idxk], bufk, sk)
            cq.wait()
            ck.wait()
            pltpu.sync_copy(bufq, qd_h.at[pl.ds(off, CH)])
            pltpu.sync_copy(bufk, xs_h.at[pl.ds(off, CH)])

    return body(qtab, ktab, dst, src)


SS = 2
NCHG = EPAD // CH
NSUP_WV = NCHG // 2 // SS
NSUP_EP = NCHG // 8 // SS


def _sc_scatter(wvT, epT, dst2, z8, z4):
    ot = (jax.ShapeDtypeStruct((2 * NPAD, 128), jnp.float32),
          jax.ShapeDtypeStruct((8 * NPAD, 32), jnp.float32))
    R8 = NPAD * 8 // 128
    R4 = NPAD * 4 // 128

    cp = pltpu.CompilerParams()
    for f, v in (("needs_layout_passes", False),
                 ("use_tc_tiling_on_sc", False)):
        if f in pltpu.CompilerParams.__dataclass_fields__:
            cp = dataclasses.replace(cp, **{f: v})

    @functools.partial(
        pl.kernel, mesh=_sc_mesh(), out_type=ot, compiler_params=cp,
        scratch_types=[pltpu.VMEM((SS, CH), jnp.int32),
                       pltpu.VMEM((8, SS * CH), jnp.float32),
                       pltpu.VMEM((8, SS * CH), jnp.float32),
                       pltpu.VMEM((R8, 128), jnp.float32),
                       pltpu.VMEM((R4, 128), jnp.float32)])
    def body(wvT_h, epT_h, dst2_h, z8_h, z4_h, accw_h, acce_h,
             idxb, bufw, bufe, acc8, acc4):
        wid = lax.axis_index("s") * NC + lax.axis_index("c")
        pltpu.sync_copy(z8_h, acc8)
        pltpu.sync_copy(z4_h, acc4)
        cvecs = [jnp.full((16,), c, jnp.int32) for c in range(8)]

        g = wid // 2
        half = wid % 2
        wv_c0 = half * (NCHG // 2)

        @pl.loop(0, NSUP_WV)
        def _(sc):
            c0 = wv_c0 + sc * SS
            pltpu.sync_copy(dst2_h.at[pl.ds(c0, SS)], idxb)
            pltpu.sync_copy(
                wvT_h.at[pl.ds(g * 8, 8), pl.ds(c0 * CH, SS * CH)], bufw)
            for j in range(SS):
                for e0 in range(8):
                    dst16 = idxb[j, pl.ds(e0 * 16, 16)]
                    q16 = lax.div(dst16, 640)
                    row16 = dst16 - q16 * 640
                    base16 = q16 * 8
                    for c in range(8):
                        vals = bufw[c, pl.ds(j * CH + e0 * 16, 16)]
                        plsc.addupdate_scatter(
                            acc8, [row16, base16 + cvecs[c]], vals)

        p = wid // 8
        eighth = wid % 8
        ep_c0 = eighth * (NCHG // 8)

        @pl.loop(0, NSUP_EP)
        def _(sc):
            c0 = ep_c0 + sc * SS
            pltpu.sync_copy(dst2_h.at[pl.ds(c0, SS)], idxb)
            pltpu.sync_copy(
                epT_h.at[pl.ds(p * 8, 8), pl.ds(c0 * CH, SS * CH)], bufe)
            for j in range(SS):
                for e0 in range(8):
                    dst16 = idxb[j, pl.ds(e0 * 16, 16)]
                    q16 = lax.div(dst16, 320)
                    row16 = dst16 - q16 * 320
                    base16 = q16 * 4
                    for c in range(4):
                        vals = bufe[c, pl.ds(j * CH + e0 * 16, 16)]
                        plsc.addupdate_scatter(
                            acc4, [row16, base16 + cvecs[c]], vals)

        for b in range(16):
            pltpu.sync_copy(
                acc8.at[:, pl.ds(8 * b, 8)],
                accw_h.at[pl.ds(half * NPAD + 640 * b, 640),
                          pl.ds(g * 8, 8)])
        for b in range(32):
            pltpu.sync_copy(
                acc4.at[:, pl.ds(4 * b, 4)],
                acce_h.at[pl.ds(eighth * NPAD + 320 * b, 320),
                          pl.ds(p * 8, 4)])

    accw, acce = body(wvT, epT, dst2, z8, z4)
    return accw.reshape(2, NPAD, 128), acce.reshape(8, NPAD, 32)


def _tc_node_transform(xin, Wq, Wkn):
    def body(x_ref, wq_ref, wk_ref, q_ref, k_ref):
        xb = x_ref[...]
        q_ref[...] = jnp.dot(xb, wq_ref[...],
                             preferred_element_type=jnp.float32)
        k_ref[...] = jnp.dot(xb, wk_ref[...],
                             preferred_element_type=jnp.float32)

    return pl.pallas_call(
        body,
        grid=(NPAD // NBLKM,),
        in_specs=[pl.BlockSpec((NBLKM, D), lambda i: (i, 0)),
                  pl.BlockSpec((D, U), lambda i: (0, 0)),
                  pl.BlockSpec((D, U), lambda i: (0, 0))],
        out_specs=[pl.BlockSpec((NBLKM, U), lambda i: (i, 0)),
                   pl.BlockSpec((NBLKM, U), lambda i: (i, 0))],
        out_shape=[jax.ShapeDtypeStruct((NPAD, U), jnp.float32),
                   jax.ShapeDtypeStruct((NPAD, U), jnp.float32)],
    )(xin, Wq, Wkn)


def _tc_edge_math(qd, xs, eap, Wke, aflat, Ssel, Bsel, hmask):
    def body(qd_ref, xs_ref, ea_ref, wke_ref, a_ref, s_ref, b_ref, hm_ref,
             wvT_ref, epT_ref):
        i = pl.program_id(0)
        k = xs_ref[...] + jnp.dot(ea_ref[...], wke_ref[...],
                                  preferred_element_type=jnp.float32)
        z = qd_ref[...] + k
        z = jnp.where(z >= 0.0, z, 0.2 * z)
        zw = z * a_ref[...]
        logits = jnp.dot(zw, s_ref[...], preferred_element_type=jnp.float32)
        e = jnp.exp(jnp.minimum(logits, 50.0)) * hm_ref[...]
        rowid = i * EBLK + lax.broadcasted_iota(jnp.int32, (EBLK, 1), 0)
        e = jnp.where(rowid < E, e, 0.0)
        wv = k * jnp.dot(e, b_ref[...], preferred_element_type=jnp.float32)
        wvT_ref[...] = wv.T
        eT = e.T
        zpad = jnp.zeros((4, EBLK), jnp.float32)
        epT_ref[...] = jnp.concatenate(
            [eT[0:4], zpad, eT[4:8], zpad, eT[8:12], zpad, eT[12:16], zpad],
            axis=0)

    return pl.pallas_call(
        body,
        grid=(EPAD // EBLK,),
        in_specs=[pl.BlockSpec((EBLK, U), lambda i: (i, 0)),
                  pl.BlockSpec((EBLK, U), lambda i: (i, 0)),
                  pl.BlockSpec((EBLK, DE), lambda i: (i, 0)),
                  pl.BlockSpec((DE, U), lambda i: (0, 0)),
                  pl.BlockSpec((1, U), lambda i: (0, 0)),
                  pl.BlockSpec((U, 16), lambda i: (0, 0)),
                  pl.BlockSpec((16, U), lambda i: (0, 0)),
                  pl.BlockSpec((1, 16), lambda i: (0, 0))],
        out_specs=[pl.BlockSpec((U, EBLK), lambda i: (0, i)),
                   pl.BlockSpec((32, EBLK), lambda i: (0, i))],
        out_shape=[jax.ShapeDtypeStruct((U, EPAD), jnp.float32),
                   jax.ShapeDtypeStruct((32, EPAD), jnp.float32)],
    )(qd, xs, eap, Wke, aflat, Ssel, Bsel, hmask)


def _tc_merge_next(accw, acce, xin, Wna, Wnb, bn, Bsel):
    def body(a0_ref, a1_ref, e0_ref, e1_ref, e2_ref, e3_ref,
             e4_ref, e5_ref, e6_ref, e7_ref,
             x_ref, wa_ref, wb_ref, b_ref, bs_ref, h_ref):
        num = a0_ref[0] + a1_ref[0]
        se = (e0_ref[0] + e1_ref[0] + e2_ref[0] + e3_ref[0]
              + e4_ref[0] + e5_ref[0] + e6_ref[0] + e7_ref[0])
        ssum = jnp.concatenate([se[:, 8 * p:8 * p + 4] for p in range(4)],
                               axis=1)
        denom = jnp.dot(ssum, bs_ref[...],
                        preferred_element_type=jnp.float32) + 1e-9
        pooled = num / denom
        h = (jnp.dot(x_ref[...], wa_ref[...],
                     preferred_element_type=jnp.float32)
             + jnp.dot(pooled, wb_ref[...],
                       preferred_element_type=jnp.float32)
             + b_ref[...])
        h_ref[...] = jnp.maximum(h, 0.0)

    especs = [pl.BlockSpec((1, NBLKM, 32), (lambda t: lambda i: (t, i, 0))(t))
              for t in range(8)]
    return pl.pallas_call(
        body,
        grid=(NPAD // NBLKM,),
        in_specs=[pl.BlockSpec((1, NBLKM, U), lambda i: (0, i, 0)),
                  pl.BlockSpec((1, NBLKM, U), lambda i: (1, i, 0))]
                 + especs
                 + [pl.BlockSpec((NBLKM, U), lambda i: (i, 0)),
                    pl.BlockSpec((U, U), lambda i: (0, 0)),
                    pl.BlockSpec((U, U), lambda i: (0, 0)),
                    pl.BlockSpec((1, U), lambda i: (0, 0)),
                    pl.BlockSpec((16, U), lambda i: (0, 0))],
        out_specs=pl.BlockSpec((NBLKM, U), lambda i: (i, 0)),
        out_shape=jax.ShapeDtypeStruct((NPAD, U), jnp.float32),
    )(accw, accw, acce, acce, acce, acce, acce, acce, acce, acce,
      xin, Wna, Wnb, bn, Bsel)


def _tc_readout(h2, Wout, bout):
    def body(h_ref, w_ref, b_ref, o_ref):
        o_ref[...] = (jnp.dot(h_ref[...], w_ref[...],
                              preferred_element_type=jnp.float32)
                      + b_ref[...])

    return pl.pallas_call(
        body,
        grid=(N // NBLK,),
        in_specs=[pl.BlockSpec((NBLK, U), lambda i: (i, 0)),
                  pl.BlockSpec((U, 1), lambda i: (0, 0)),
                  pl.BlockSpec((1, 1), lambda i: (0, 0))],
        out_specs=pl.BlockSpec((NBLK, 1), lambda i: (i, 0)),
        out_shape=jax.ShapeDtypeStruct((N, 1), jnp.float32),
    )(h2, Wout, bout)


def _gat_layer(xin, Wq, Wk, aflat, Ssel, Bsel, hmask, Wn, bn,
               dstp, srcp, dst2, eap, z8, z4, din):
    Wkn = Wk[:din]
    Wke = Wk[din:]
    qtab, ktab = _tc_node_transform(xin, Wq, Wkn)
    qd, xs = _sc_gather(qtab, ktab, dstp, srcp)
    wvT, epT = _tc_edge_math(qd, xs, eap, Wke, aflat, Ssel, Bsel, hmask)
    accw, acce = _sc_scatter(wvT, epT, dst2, z8, z4)
    Wna = Wn[:din]
    Wnb = Wn[din:]
    return _tc_merge_next(accw, acce, xin, Wna, Wnb, bn.reshape(1, U), Bsel)


def kernel(x, edge_index, edge_attr, Wq1, Wk1, a1, Wn1, bn1,
           Wq2, Wk2, a2, Wn2, bn2, Wout, bout):
    src = edge_index[0]
    dst = edge_index[1]
    pad = EPAD - E
    dstp = jnp.concatenate([dst, jnp.zeros((pad,), dst.dtype)])
    srcp = jnp.concatenate([src, jnp.zeros((pad,), src.dtype)])
    eap = jnp.concatenate(
        [edge_attr, jnp.zeros((pad, DE), edge_attr.dtype)], axis=0)
    dst2 = dstp.reshape(EPAD // CH, CH)
    xpad = jnp.concatenate(
        [x, jnp.zeros((NPAD - N, D), jnp.float32)], axis=0)
    z8 = jnp.zeros((NPAD * 8 // 128, 128), jnp.float32)
    z4 = jnp.zeros((NPAD * 4 // 128, 128), jnp.float32)

    lanes = jnp.arange(16)
    head1 = jnp.arange(U) // 32
    S1 = (head1[:, None] == lanes[None, :]).astype(jnp.float32)
    B1 = S1.T
    hm1 = (lanes < 4).astype(jnp.float32).reshape(1, 16)
    S2 = (jnp.zeros((U, 1), jnp.int32) == lanes[None, :]).astype(jnp.float32)
    B2 = S2.T
    hm2 = (lanes < 1).astype(jnp.float32).reshape(1, 16)

    a1flat = a1.reshape(1, U)
    a2flat = a2.reshape(1, U)

    h1 = _gat_layer(xpad, Wq1, Wk1, a1flat, S1, B1, hm1, Wn1, bn1,
                    dstp, srcp, dst2, eap, z8, z4, D)
    h2 = _gat_layer(h1, Wq2, Wk2, a2flat, S2, B2, hm2, Wn2, bn2,
                    dstp, srcp, dst2, eap, z8, z4, U)
    return _tc_readout(h2, Wout, bout.reshape(1, 1))


# speedup vs baseline: 6.5228x; 1.1299x over previous
import dataclasses
import functools

import jax
import jax.numpy as jnp
from jax import lax
from jax.experimental import pallas as pl
from jax.experimental.pallas import tpu as pltpu
from jax.experimental.pallas import tpu_sc as plsc

N = 10000
E = 320000
D = 128
DE = 16
U = 128

NC = 2
NS = 16
NW = NC * NS
CH = 128
NCHUNK = -(-E // (NW * CH))
EPAD = NW * CH * NCHUNK
PERW = CH * NCHUNK
NPAD = 10240
RPS = NPAD // NS

EBLK = 512
NBLKM = 1280
NBLK = 1000


def _sc_mesh():
    return plsc.VectorSubcoreMesh(core_axis_name="c", subcore_axis_name="s")


def _sc_gather(qtab, ktab, dst, src):
    ot = (jax.ShapeDtypeStruct((EPAD, U), jnp.float32),
          jax.ShapeDtypeStruct((EPAD, U), jnp.float32))

    @functools.partial(
        pl.kernel, mesh=_sc_mesh(), out_type=ot,
        scratch_types=[pltpu.VMEM((CH,), jnp.int32),
                       pltpu.VMEM((CH,), jnp.int32),
                       pltpu.VMEM((CH, U), jnp.float32),
                       pltpu.VMEM((CH, U), jnp.float32),
                       pltpu.SemaphoreType.DMA,
                       pltpu.SemaphoreType.DMA])
    def body(qtab_h, ktab_h, dst_h, src_h, qd_h, xs_h,
             idxq, idxk, bufq, bufk, sq, sk):
        wid = lax.axis_index("s") * NC + lax.axis_index("c")
        base = wid * PERW

        @pl.loop(0, NCHUNK)
        def _(j):
            off = base + j * CH
            pltpu.sync_copy(dst_h.at[pl.ds(off, CH)], idxq)
            pltpu.sync_copy(src_h.at[pl.ds(off, CH)], idxk)
            cq = pltpu.async_copy(qtab_h.at[idxq], bufq, sq)
            ck = pltpu.async_copy(ktab_h.at[idxk], bufk, sk)
            cq.wait()
            ck.wait()
            pltpu.sync_copy(bufq, qd_h.at[pl.ds(off, CH)])
            pltpu.sync_copy(bufk, xs_h.at[pl.ds(off, CH)])

    return body(qtab, ktab, dst, src)


SS = 2
NCHG = EPAD // CH
NSUP_WV = NCHG // 2 // SS
NSUP_EP = NCHG // 8 // SS


def _sc_scatter(wvT, epT, dst2, z8, z4):
    ot = (jax.ShapeDtypeStruct((2 * NPAD, 128), jnp.float32),
          jax.ShapeDtypeStruct((8 * NPAD, 32), jnp.float32))

    cp = pltpu.CompilerParams()
    for f, v in (("needs_layout_passes", False),
                 ("use_tc_tiling_on_sc", False)):
        if f in pltpu.CompilerParams.__dataclass_fields__:
            cp = dataclasses.replace(cp, **{f: v})

    @functools.partial(
        pl.kernel, mesh=_sc_mesh(), out_type=ot, compiler_params=cp,
        scratch_types=[pltpu.VMEM((SS, CH), jnp.int32),
                       pltpu.VMEM((SS, CH), jnp.int32),
                       pltpu.VMEM((8, SS * CH), jnp.float32),
                       pltpu.VMEM((8, SS * CH), jnp.float32),
                       pltpu.VMEM((NPAD * 8 // 128, 128), jnp.float32),
                       pltpu.VMEM((NPAD * 4 // 128, 128), jnp.float32),
                       pltpu.SemaphoreType.DMA,
                       pltpu.SemaphoreType.DMA])
    def body(wvT_h, epT_h, dst2_h, z8_h, z4_h, accw_h, acce_h,
             idxb0, idxb1, buf0, buf1, acc8, acc4, sem0, sem1):
        wid = lax.axis_index("s") * NC + lax.axis_index("c")
        pltpu.sync_copy(z8_h, acc8)
        pltpu.sync_copy(z4_h, acc4)
        cvecs = [jnp.full((16,), c, jnp.int32) for c in range(8)]
        idxbs = (idxb0, idxb1)
        bufs = (buf0, buf1)
        sems = (sem0, sem1)

        g = wid // 2
        half = wid % 2
        p = wid // 8
        eighth = wid % 8

        def run_phase(src_h, row0, c0base, climit, nsup, acc, ncols,
                      divisor, gwidth):
            def start(k, c0raw):
                c0 = jnp.minimum(c0raw, climit)
                pltpu.async_copy(dst2_h.at[pl.ds(c0, SS)], idxbs[k],
                                 sems[k])
                pltpu.async_copy(
                    src_h.at[pl.ds(row0, 8), pl.ds(c0 * CH, SS * CH)],
                    bufs[k], sems[k])

            def wait(k):
                pltpu.make_async_copy(dst2_h.at[pl.ds(0, SS)], idxbs[k],
                                      sems[k]).wait()
                pltpu.make_async_copy(
                    src_h.at[pl.ds(0, 8), pl.ds(0, SS * CH)],
                    bufs[k], sems[k]).wait()

            def scatter(k):
                for j in range(SS):
                    for e0 in range(8):
                        dst16 = idxbs[k][j, pl.ds(e0 * 16, 16)]
                        q16 = lax.div(dst16, divisor)
                        row16 = dst16 - q16 * divisor
                        base16 = q16 * gwidth
                        for c in range(ncols):
                            vals = bufs[k][c, pl.ds(j * CH + e0 * 16, 16)]
                            plsc.addupdate_scatter(
                                acc, [row16, base16 + cvecs[c]], vals)

            start(0, c0base)
            start(1, c0base + SS)

            @pl.loop(0, nsup // 2)
            def _(it):
                c0 = c0base + it * (2 * SS)
                wait(0)
                scatter(0)
                start(0, c0 + 2 * SS)
                wait(1)
                scatter(1)
                start(1, c0 + 3 * SS)

            wait(0)
            wait(1)

        wv_c0 = half * (NCHG // 2)
        run_phase(wvT_h, g * 8, wv_c0, wv_c0 + NCHG // 2 - SS, NSUP_WV,
                  acc8, 8, 640, 8)
        ep_c0 = eighth * (NCHG // 8)
        run_phase(epT_h, p * 8, ep_c0, ep_c0 + NCHG // 8 - SS, NSUP_EP,
                  acc4, 4, 320, 4)

        for b in range(16):
            pltpu.sync_copy(
                acc8.at[:, pl.ds(8 * b, 8)],
                accw_h.at[pl.ds(half * NPAD + 640 * b, 640),
                          pl.ds(g * 8, 8)])
        for b in range(32):
            pltpu.sync_copy(
                acc4.at[:, pl.ds(4 * b, 4)],
                acce_h.at[pl.ds(eighth * NPAD + 320 * b, 320),
                          pl.ds(p * 8, 4)])

    accw, acce = body(wvT, epT, dst2, z8, z4)
    return accw.reshape(2, NPAD, 128), acce.reshape(8, NPAD, 32)


def _tc_node_transform(xin, Wq, Wkn):
    def body(x_ref, wq_ref, wk_ref, q_ref, k_ref):
        xb = x_ref[...]
        q_ref[...] = jnp.dot(xb, wq_ref[...],
                             preferred_element_type=jnp.float32)
        k_ref[...] = jnp.dot(xb, wk_ref[...],
                             preferred_element_type=jnp.float32)

    return pl.pallas_call(
        body,
        grid=(NPAD // NBLKM,),
        in_specs=[pl.BlockSpec((NBLKM, D), lambda i: (i, 0)),
                  pl.BlockSpec((D, U), lambda i: (0, 0)),
                  pl.BlockSpec((D, U), lambda i: (0, 0))],
        out_specs=[pl.BlockSpec((NBLKM, U), lambda i: (i, 0)),
                   pl.BlockSpec((NBLKM, U), lambda i: (i, 0))],
        out_shape=[jax.ShapeDtypeStruct((NPAD, U), jnp.float32),
                   jax.ShapeDtypeStruct((NPAD, U), jnp.float32)],
    )(xin, Wq, Wkn)


def _tc_edge_math(qd, xs, eap, Wke, aflat, Ssel, Bsel, hmask):
    def body(qd_ref, xs_ref, ea_ref, wke_ref, a_ref, s_ref, b_ref, hm_ref,
             wvT_ref, epT_ref):
        i = pl.program_id(0)
        k = xs_ref[...] + jnp.dot(ea_ref[...], wke_ref[...],
                                  preferred_element_type=jnp.float32)
        z = qd_ref[...] + k
        z = jnp.where(z >= 0.0, z, 0.2 * z)
        zw = z * a_ref[...]
        logits = jnp.dot(zw, s_ref[...], preferred_element_type=jnp.float32)
        e = jnp.exp(jnp.minimum(logits, 50.0)) * hm_ref[...]
        rowid = i * EBLK + lax.broadcasted_iota(jnp.int32, (EBLK, 1), 0)
        e = jnp.where(rowid < E, e, 0.0)
        wv = k * jnp.dot(e, b_ref[...], preferred_element_type=jnp.float32)
        wvT_ref[...] = wv.T
        eT = e.T
        zpad = jnp.zeros((4, EBLK), jnp.float32)
        epT_ref[...] = jnp.concatenate(
            [eT[0:4], zpad, eT[4:8], zpad, eT[8:12], zpad, eT[12:16], zpad],
            axis=0)

    return pl.pallas_call(
        body,
        grid=(EPAD // EBLK,),
        in_specs=[pl.BlockSpec((EBLK, U), lambda i: (i, 0)),
                  pl.BlockSpec((EBLK, U), lambda i: (i, 0)),
                  pl.BlockSpec((EBLK, DE), lambda i: (i, 0)),
                  pl.BlockSpec((DE, U), lambda i: (0, 0)),
                  pl.BlockSpec((1, U), lambda i: (0, 0)),
                  pl.BlockSpec((U, 16), lambda i: (0, 0)),
                  pl.BlockSpec((16, U), lambda i: (0, 0)),
                  pl.BlockSpec((1, 16), lambda i: (0, 0))],
        out_specs=[pl.BlockSpec((U, EBLK), lambda i: (0, i)),
                   pl.BlockSpec((32, EBLK), lambda i: (0, i))],
        out_shape=[jax.ShapeDtypeStruct((U, EPAD), jnp.float32),
                   jax.ShapeDtypeStruct((32, EPAD), jnp.float32)],
    )(qd, xs, eap, Wke, aflat, Ssel, Bsel, hmask)


def _tc_merge_next(accw, acce, xin, Wna, Wnb, bn, Bsel):
    def body(a0_ref, a1_ref, e0_ref, e1_ref, e2_ref, e3_ref,
             e4_ref, e5_ref, e6_ref, e7_ref,
             x_ref, wa_ref, wb_ref, b_ref, bs_ref, h_ref):
        num = a0_ref[0] + a1_ref[0]
        se = (e0_ref[0] + e1_ref[0] + e2_ref[0] + e3_ref[0]
              + e4_ref[0] + e5_ref[0] + e6_ref[0] + e7_ref[0])
        ssum = jnp.concatenate([se[:, 8 * p:8 * p + 4] for p in range(4)],
                               axis=1)
        denom = jnp.dot(ssum, bs_ref[...],
                        preferred_element_type=jnp.float32) + 1e-9
        pooled = num / denom
        h = (jnp.dot(x_ref[...], wa_ref[...],
                     preferred_element_type=jnp.float32)
             + jnp.dot(pooled, wb_ref[...],
                       preferred_element_type=jnp.float32)
             + b_ref[...])
        h_ref[...] = jnp.maximum(h, 0.0)

    especs = [pl.BlockSpec((1, NBLKM, 32), (lambda t: lambda i: (t, i, 0))(t))
              for t in range(8)]
    return pl.pallas_call(
        body,
        grid=(NPAD // NBLKM,),
        in_specs=[pl.BlockSpec((1, NBLKM, U), lambda i: (0, i, 0)),
                  pl.BlockSpec((1, NBLKM, U), lambda i: (1, i, 0))]
                 + especs
                 + [pl.BlockSpec((NBLKM, U), lambda i: (i, 0)),
                    pl.BlockSpec((U, U), lambda i: (0, 0)),
                    pl.BlockSpec((U, U), lambda i: (0, 0)),
                    pl.BlockSpec((1, U), lambda i: (0, 0)),
                    pl.BlockSpec((16, U), lambda i: (0, 0))],
        out_specs=pl.BlockSpec((NBLKM, U), lambda i: (i, 0)),
        out_shape=jax.ShapeDtypeStruct((NPAD, U), jnp.float32),
    )(accw, accw, acce, acce, acce, acce, acce, acce, acce, acce,
      xin, Wna, Wnb, bn, Bsel)


def _tc_readout(h2, Wout, bout):
    def body(h_ref, w_ref, b_ref, o_ref):
        o_ref[...] = (jnp.dot(h_ref[...], w_ref[...],
                              preferred_element_type=jnp.float32)
                      + b_ref[...])

    return pl.pallas_call(
        body,
        grid=(N // NBLK,),
        in_specs=[pl.BlockSpec((NBLK, U), lambda i: (i, 0)),
                  pl.BlockSpec((U, 1), lambda i: (0, 0)),
                  pl.BlockSpec((1, 1), lambda i: (0, 0))],
        out_specs=pl.BlockSpec((NBLK, 1), lambda i: (i, 0)),
        out_shape=jax.ShapeDtypeStruct((N, 1), jnp.float32),
    )(h2, Wout, bout)


def _gat_layer(xin, Wq, Wk, aflat, Ssel, Bsel, hmask, Wn, bn,
               dstp, srcp, dst2, eap, z8, z4, din):
    Wkn = Wk[:din]
    Wke = Wk[din:]
    qtab, ktab = _tc_node_transform(xin, Wq, Wkn)
    qd, xs = _sc_gather(qtab, ktab, dstp, srcp)
    wvT, epT = _tc_edge_math(qd, xs, eap, Wke, aflat, Ssel, Bsel, hmask)
    accw, acce = _sc_scatter(wvT, epT, dst2, z8, z4)
    Wna = Wn[:din]
    Wnb = Wn[din:]
    return _tc_merge_next(accw, acce, xin, Wna, Wnb, bn.reshape(1, U), Bsel)


def kernel(x, edge_index, edge_attr, Wq1, Wk1, a1, Wn1, bn1,
           Wq2, Wk2, a2, Wn2, bn2, Wout, bout):
    src = edge_index[0]
    dst = edge_index[1]
    pad = EPAD - E
    dstp = jnp.concatenate([dst, jnp.zeros((pad,), dst.dtype)])
    srcp = jnp.concatenate([src, jnp.zeros((pad,), src.dtype)])
    eap = jnp.concatenate(
        [edge_attr, jnp.zeros((pad, DE), edge_attr.dtype)], axis=0)
    dst2 = dstp.reshape(EPAD // CH, CH)
    xpad = jnp.concatenate(
        [x, jnp.zeros((NPAD - N, D), jnp.float32)], axis=0)
    z8 = jnp.zeros((NPAD * 8 // 128, 128), jnp.float32)
    z4 = jnp.zeros((NPAD * 4 // 128, 128), jnp.float32)

    lanes = jnp.arange(16)
    head1 = jnp.arange(U) // 32
    S1 = (head1[:, None] == lanes[None, :]).astype(jnp.float32)
    B1 = S1.T
    hm1 = (lanes < 4).astype(jnp.float32).reshape(1, 16)
    S2 = (jnp.zeros((U, 1), jnp.int32) == lanes[None, :]).astype(jnp.float32)
    B2 = S2.T
    hm2 = (lanes < 1).astype(jnp.float32).reshape(1, 16)

    a1flat = a1.reshape(1, U)
    a2flat = a2.reshape(1, U)

    h1 = _gat_layer(xpad, Wq1, Wk1, a1flat, S1, B1, hm1, Wn1, bn1,
                    dstp, srcp, dst2, eap, z8, z4, D)
    h2 = _gat_layer(h1, Wq2, Wk2, a2flat, S2, B2, hm2, Wn2, bn2,
                    dstp, srcp, dst2, eap, z8, z4, U)
    return _tc_readout(h2, Wout, bout.reshape(1, 1))


# speedup vs baseline: 6.5934x; 1.0108x over previous
import dataclasses
import functools

import jax
import jax.numpy as jnp
from jax import lax
from jax.experimental import pallas as pl
from jax.experimental.pallas import tpu as pltpu
from jax.experimental.pallas import tpu_sc as plsc

N = 10000
E = 320000
D = 128
DE = 16
U = 128

NC = 2
NS = 16
NW = NC * NS
CH = 128
NCHUNK = -(-E // (NW * CH))
EPAD = NW * CH * NCHUNK
PERW = CH * NCHUNK
NPAD = 10240
RPS = NPAD // NS

EBLK = 512
NBLKM = 1280
NBLK = 1000


def _sc_mesh():
    return plsc.VectorSubcoreMesh(core_axis_name="c", subcore_axis_name="s")


def _sc_gather(qtab, ktab, dst, src):
    ot = (jax.ShapeDtypeStruct((EPAD, U), jnp.float32),
          jax.ShapeDtypeStruct((EPAD, U), jnp.float32))

    @functools.partial(
        pl.kernel, mesh=_sc_mesh(), out_type=ot,
        scratch_types=[pltpu.VMEM((CH,), jnp.int32),
                       pltpu.VMEM((CH,), jnp.int32),
                       pltpu.VMEM((CH,), jnp.int32),
                       pltpu.VMEM((CH,), jnp.int32),
                       pltpu.VMEM((CH, U), jnp.float32),
                       pltpu.VMEM((CH, U), jnp.float32),
                       pltpu.VMEM((CH, U), jnp.float32),
                       pltpu.VMEM((CH, U), jnp.float32),
                       pltpu.SemaphoreType.DMA,
                       pltpu.SemaphoreType.DMA,
                       pltpu.SemaphoreType.DMA,
                       pltpu.SemaphoreType.DMA])
    def body(qtab_h, ktab_h, dst_h, src_h, qd_h, xs_h,
             idxq0, idxk0, idxq1, idxk1, bufq0, bufk0, bufq1, bufk1,
             si0, si1, sg0, sg1):
        wid = lax.axis_index("s") * NC + lax.axis_index("c")
        base = wid * PERW
        lim = base + PERW - CH
        idxqs = (idxq0, idxq1)
        idxks = (idxk0, idxk1)
        bufqs = (bufq0, bufq1)
        bufks = (bufk0, bufk1)
        sis = (si0, si1)
        sgs = (sg0, sg1)

        def start_idx(k, offraw):
            off = jnp.minimum(offraw, lim)
            pltpu.async_copy(dst_h.at[pl.ds(off, CH)], idxqs[k], sis[k])
            pltpu.async_copy(src_h.at[pl.ds(off, CH)], idxks[k], sis[k])

        def wait_idx(k):
            pltpu.make_async_copy(dst_h.at[pl.ds(0, CH)], idxqs[k],
                                  sis[k]).wait()
            pltpu.make_async_copy(src_h.at[pl.ds(0, CH)], idxks[k],
                                  sis[k]).wait()

        def start_gather(k):
            pltpu.async_copy(qtab_h.at[idxqs[k]], bufqs[k], sgs[k])
            pltpu.async_copy(ktab_h.at[idxks[k]], bufks[k], sgs[k])

        def wait_gather(k):
            pltpu.make_async_copy(qtab_h.at[pl.ds(0, CH)], bufqs[k],
                                  sgs[k]).wait()
            pltpu.make_async_copy(ktab_h.at[pl.ds(0, CH)], bufks[k],
                                  sgs[k]).wait()

        start_idx(0, base)
        start_idx(1, base + CH)

        @pl.loop(0, NCHUNK // 2)
        def _(it):
            off = base + it * (2 * CH)
            wait_idx(0)
            start_gather(0)
            wait_idx(1)
            start_gather(1)
            wait_gather(0)
            pltpu.sync_copy(bufqs[0], qd_h.at[pl.ds(off, CH)])
            pltpu.sync_copy(bufks[0], xs_h.at[pl.ds(off, CH)])
            start_idx(0, off + 2 * CH)
            wait_gather(1)
            pltpu.sync_copy(bufqs[1], qd_h.at[pl.ds(off + CH, CH)])
            pltpu.sync_copy(bufks[1], xs_h.at[pl.ds(off + CH, CH)])
            start_idx(1, off + 3 * CH)

        wait_idx(0)
        wait_idx(1)

        @pl.when(NCHUNK % 2 == 1)
        def _():
            off = base + (NCHUNK - 1) * CH
            pltpu.sync_copy(dst_h.at[pl.ds(off, CH)], idxq0)
            pltpu.sync_copy(src_h.at[pl.ds(off, CH)], idxk0)
            cq = pltpu.async_copy(qtab_h.at[idxq0], bufq0, sg0)
            ck = pltpu.async_copy(ktab_h.at[idxk0], bufk0, sg1)
            cq.wait()
            ck.wait()
            pltpu.sync_copy(bufq0, qd_h.at[pl.ds(off, CH)])
            pltpu.sync_copy(bufk0, xs_h.at[pl.ds(off, CH)])

    return body(qtab, ktab, dst, src)


SS = 2
NCHG = EPAD // CH
NSUP_WV = NCHG // 2 // SS
NSUP_EP = NCHG // 8 // SS


def _sc_scatter(wvT, epT, dst2, z8, z4):
    ot = (jax.ShapeDtypeStruct((2 * NPAD, 128), jnp.float32),
          jax.ShapeDtypeStruct((8 * NPAD, 32), jnp.float32))

    cp = pltpu.CompilerParams()
    for f, v in (("needs_layout_passes", False),
                 ("use_tc_tiling_on_sc", False)):
        if f in pltpu.CompilerParams.__dataclass_fields__:
            cp = dataclasses.replace(cp, **{f: v})

    @functools.partial(
        pl.kernel, mesh=_sc_mesh(), out_type=ot, compiler_params=cp,
        scratch_types=[pltpu.VMEM((SS, CH), jnp.int32),
                       pltpu.VMEM((SS, CH), jnp.int32),
                       pltpu.VMEM((8, SS * CH), jnp.float32),
                       pltpu.VMEM((8, SS * CH), jnp.float32),
                       pltpu.VMEM((NPAD * 8 // 128, 128), jnp.float32),
                       pltpu.VMEM((NPAD * 4 // 128, 128), jnp.float32),
                       pltpu.SemaphoreType.DMA,
                       pltpu.SemaphoreType.DMA])
    def body(wvT_h, epT_h, dst2_h, z8_h, z4_h, accw_h, acce_h,
             idxb0, idxb1, buf0, buf1, acc8, acc4, sem0, sem1):
        wid = lax.axis_index("s") * NC + lax.axis_index("c")
        pltpu.sync_copy(z8_h, acc8)
        pltpu.sync_copy(z4_h, acc4)
        cvecs = [jnp.full((16,), c, jnp.int32) for c in range(8)]
        idxbs = (idxb0, idxb1)
        bufs = (buf0, buf1)
        sems = (sem0, sem1)

        g = wid // 2
        half = wid % 2
        p = wid // 8
        eighth = wid % 8

        def run_phase(src_h, row0, c0base, climit, nsup, acc, ncols,
                      divisor, gwidth):
            def start(k, c0raw):
                c0 = jnp.minimum(c0raw, climit)
                pltpu.async_copy(dst2_h.at[pl.ds(c0, SS)], idxbs[k],
                                 sems[k])
                pltpu.async_copy(
                    src_h.at[pl.ds(row0, 8), pl.ds(c0 * CH, SS * CH)],
                    bufs[k], sems[k])

            def wait(k):
                pltpu.make_async_copy(dst2_h.at[pl.ds(0, SS)], idxbs[k],
                                      sems[k]).wait()
                pltpu.make_async_copy(
                    src_h.at[pl.ds(0, 8), pl.ds(0, SS * CH)],
                    bufs[k], sems[k]).wait()

            def scatter(k):
                for j in range(SS):
                    for e0 in range(8):
                        dst16 = idxbs[k][j, pl.ds(e0 * 16, 16)]
                        q16 = lax.div(dst16, divisor)
                        row16 = dst16 - q16 * divisor
                        base16 = q16 * gwidth
                        for c in range(ncols):
                            vals = bufs[k][c, pl.ds(j * CH + e0 * 16, 16)]
                            plsc.addupdate_scatter(
                                acc, [row16, base16 + cvecs[c]], vals)

            start(0, c0base)
            start(1, c0base + SS)

            @pl.loop(0, nsup // 2)
            def _(it):
                c0 = c0base + it * (2 * SS)
                wait(0)
                scatter(0)
                start(0, c0 + 2 * SS)
                wait(1)
                scatter(1)
                start(1, c0 + 3 * SS)

            wait(0)
            wait(1)

        wv_c0 = half * (NCHG // 2)
        run_phase(wvT_h, g * 8, wv_c0, wv_c0 + NCHG // 2 - SS, NSUP_WV,
                  acc8, 8, 640, 8)
        ep_c0 = eighth * (NCHG // 8)
        run_phase(epT_h, p * 8, ep_c0, ep_c0 + NCHG // 8 - SS, NSUP_EP,
                  acc4, 4, 320, 4)

        for b in range(16):
            pltpu.sync_copy(
                acc8.at[:, pl.ds(8 * b, 8)],
                accw_h.at[pl.ds(half * NPAD + 640 * b, 640),
                          pl.ds(g * 8, 8)])
        for b in range(32):
            pltpu.sync_copy(
                acc4.at[:, pl.ds(4 * b, 4)],
                acce_h.at[pl.ds(eighth * NPAD + 320 * b, 320),
                          pl.ds(p * 8, 4)])

    accw, acce = body(wvT, epT, dst2, z8, z4)
    return accw.reshape(2, NPAD, 128), acce.reshape(8, NPAD, 32)


def _tc_node_transform(xin, Wq, Wkn):
    def body(x_ref, wq_ref, wk_ref, q_ref, k_ref):
        xb = x_ref[...]
        q_ref[...] = jnp.dot(xb, wq_ref[...],
                             preferred_element_type=jnp.float32)
        k_ref[...] = jnp.dot(xb, wk_ref[...],
                             preferred_element_type=jnp.float32)

    return pl.pallas_call(
        body,
        grid=(NPAD // NBLKM,),
        in_specs=[pl.BlockSpec((NBLKM, D), lambda i: (i, 0)),
                  pl.BlockSpec((D, U), lambda i: (0, 0)),
                  pl.BlockSpec((D, U), lambda i: (0, 0))],
        out_specs=[pl.BlockSpec((NBLKM, U), lambda i: (i, 0)),
                   pl.BlockSpec((NBLKM, U), lambda i: (i, 0))],
        out_shape=[jax.ShapeDtypeStruct((NPAD, U), jnp.float32),
                   jax.ShapeDtypeStruct((NPAD, U), jnp.float32)],
    )(xin, Wq, Wkn)


def _tc_edge_math(qd, xs, eap, Wke, aflat, Ssel, Bsel, hmask):
    def body(qd_ref, xs_ref, ea_ref, wke_ref, a_ref, s_ref, b_ref, hm_ref,
             wvT_ref, epT_ref):
        i = pl.program_id(0)
        k = xs_ref[...] + jnp.dot(ea_ref[...], wke_ref[...],
                                  preferred_element_type=jnp.float32)
        z = qd_ref[...] + k
        z = jnp.where(z >= 0.0, z, 0.2 * z)
        zw = z * a_ref[...]
        logits = jnp.dot(zw, s_ref[...], preferred_element_type=jnp.float32)
        e = jnp.exp(jnp.minimum(logits, 50.0)) * hm_ref[...]
        rowid = i * EBLK + lax.broadcasted_iota(jnp.int32, (EBLK, 1), 0)
        e = jnp.where(rowid < E, e, 0.0)
        wv = k * jnp.dot(e, b_ref[...], preferred_element_type=jnp.float32)
        wvT_ref[...] = wv.T
        eT = e.T
        zpad = jnp.zeros((4, EBLK), jnp.float32)
        epT_ref[...] = jnp.concatenate(
            [eT[0:4], zpad, eT[4:8], zpad, eT[8:12], zpad, eT[12:16], zpad],
            axis=0)

    return pl.pallas_call(
        body,
        grid=(EPAD // EBLK,),
        in_specs=[pl.BlockSpec((EBLK, U), lambda i: (i, 0)),
                  pl.BlockSpec((EBLK, U), lambda i: (i, 0)),
                  pl.BlockSpec((EBLK, DE), lambda i: (i, 0)),
                  pl.BlockSpec((DE, U), lambda i: (0, 0)),
                  pl.BlockSpec((1, U), lambda i: (0, 0)),
                  pl.BlockSpec((U, 16), lambda i: (0, 0)),
                  pl.BlockSpec((16, U), lambda i: (0, 0)),
                  pl.BlockSpec((1, 16), lambda i: (0, 0))],
        out_specs=[pl.BlockSpec((U, EBLK), lambda i: (0, i)),
                   pl.BlockSpec((32, EBLK), lambda i: (0, i))],
        out_shape=[jax.ShapeDtypeStruct((U, EPAD), jnp.float32),
                   jax.ShapeDtypeStruct((32, EPAD), jnp.float32)],
    )(qd, xs, eap, Wke, aflat, Ssel, Bsel, hmask)


def _tc_merge_next(accw, acce, xin, Wna, Wnb, bn, Bsel):
    def body(a0_ref, a1_ref, e0_ref, e1_ref, e2_ref, e3_ref,
             e4_ref, e5_ref, e6_ref, e7_ref,
             x_ref, wa_ref, wb_ref, b_ref, bs_ref, h_ref):
        num = a0_ref[0] + a1_ref[0]
        se = (e0_ref[0] + e1_ref[0] + e2_ref[0] + e3_ref[0]
              + e4_ref[0] + e5_ref[0] + e6_ref[0] + e7_ref[0])
        ssum = jnp.concatenate([se[:, 8 * p:8 * p + 4] for p in range(4)],
                               axis=1)
        denom = jnp.dot(ssum, bs_ref[...],
                        preferred_element_type=jnp.float32) + 1e-9
        pooled = num / denom
        h = (jnp.dot(x_ref[...], wa_ref[...],
                     preferred_element_type=jnp.float32)
             + jnp.dot(pooled, wb_ref[...],
                       preferred_element_type=jnp.float32)
             + b_ref[...])
        h_ref[...] = jnp.maximum(h, 0.0)

    especs = [pl.BlockSpec((1, NBLKM, 32), (lambda t: lambda i: (t, i, 0))(t))
              for t in range(8)]
    return pl.pallas_call(
        body,
        grid=(NPAD // NBLKM,),
        in_specs=[pl.BlockSpec((1, NBLKM, U), lambda i: (0, i, 0)),
                  pl.BlockSpec((1, NBLKM, U), lambda i: (1, i, 0))]
                 + especs
                 + [pl.BlockSpec((NBLKM, U), lambda i: (i, 0)),
                    pl.BlockSpec((U, U), lambda i: (0, 0)),
                    pl.BlockSpec((U, U), lambda i: (0, 0)),
                    pl.BlockSpec((1, U), lambda i: (0, 0)),
                    pl.BlockSpec((16, U), lambda i: (0, 0))],
        out_specs=pl.BlockSpec((NBLKM, U), lambda i: (i, 0)),
        out_shape=jax.ShapeDtypeStruct((NPAD, U), jnp.float32),
    )(accw, accw, acce, acce, acce, acce, acce, acce, acce, acce,
      xin, Wna, Wnb, bn, Bsel)


def _tc_readout(h2, Wout, bout):
    def body(h_ref, w_ref, b_ref, o_ref):
        o_ref[...] = (jnp.dot(h_ref[...], w_ref[...],
                              preferred_element_type=jnp.float32)
                      + b_ref[...])

    return pl.pallas_call(
        body,
        grid=(N // NBLK,),
        in_specs=[pl.BlockSpec((NBLK, U), lambda i: (i, 0)),
                  pl.BlockSpec((U, 1), lambda i: (0, 0)),
                  pl.BlockSpec((1, 1), lambda i: (0, 0))],
        out_specs=pl.BlockSpec((NBLK, 1), lambda i: (i, 0)),
        out_shape=jax.ShapeDtypeStruct((N, 1), jnp.float32),
    )(h2, Wout, bout)


def _gat_layer(xin, Wq, Wk, aflat, Ssel, Bsel, hmask, Wn, bn,
               dstp, srcp, dst2, eap, z8, z4, din):
    Wkn = Wk[:din]
    Wke = Wk[din:]
    qtab, ktab = _tc_node_transform(xin, Wq, Wkn)
    qd, xs = _sc_gather(qtab, ktab, dstp, srcp)
    wvT, epT = _tc_edge_math(qd, xs, eap, Wke, aflat, Ssel, Bsel, hmask)
    accw, acce = _sc_scatter(wvT, epT, dst2, z8, z4)
    Wna = Wn[:din]
    Wnb = Wn[din:]
    return _tc_merge_next(accw, acce, xin, Wna, Wnb, bn.reshape(1, U), Bsel)


def kernel(x, edge_index, edge_attr, Wq1, Wk1, a1, Wn1, bn1,
           Wq2, Wk2, a2, Wn2, bn2, Wout, bout):
    src = edge_index[0]
    dst = edge_index[1]
    pad = EPAD - E
    dstp = jnp.concatenate([dst, jnp.zeros((pad,), dst.dtype)])
    srcp = jnp.concatenate([src, jnp.zeros((pad,), src.dtype)])
    eap = jnp.concatenate(
        [edge_attr, jnp.zeros((pad, DE), edge_attr.dtype)], axis=0)
    dst2 = dstp.reshape(EPAD // CH, CH)
    xpad = jnp.concatenate(
        [x, jnp.zeros((NPAD - N, D), jnp.float32)], axis=0)
    z8 = jnp.zeros((NPAD * 8 // 128, 128), jnp.float32)
    z4 = jnp.zeros((NPAD * 4 // 128, 128), jnp.float32)

    lanes = jnp.arange(16)
    head1 = jnp.arange(U) // 32
    S1 = (head1[:, None] == lanes[None, :]).astype(jnp.float32)
    B1 = S1.T
    hm1 = (lanes < 4).astype(jnp.float32).reshape(1, 16)
    S2 = (jnp.zeros((U, 1), jnp.int32) == lanes[None, :]).astype(jnp.float32)
    B2 = S2.T
    hm2 = (lanes < 1).astype(jnp.float32).reshape(1, 16)

    a1flat = a1.reshape(1, U)
    a2flat = a2.reshape(1, U)

    h1 = _gat_layer(xpad, Wq1, Wk1, a1flat, S1, B1, hm1, Wn1, bn1,
                    dstp, srcp, dst2, eap, z8, z4, D)
    h2 = _gat_layer(h1, Wq2, Wk2, a2flat, S2, B2, hm2, Wn2, bn2,
                    dstp, srcp, dst2, eap, z8, z4, U)
    return _tc_readout(h2, Wout, bout.reshape(1, 1))
